# baseline (speedup 1.0000x reference)
import jax
import jax.numpy as jnp
from jax.experimental import pallas as pl
from jax.experimental.pallas import tpu as pltpu

_P = 4
_TB = 4096
_G = 8


def _head_kernel(x_ref, w_ref, b_ref, pi_ref, vf_ref):
    cols = pi_ref.shape[1]
    out = jnp.dot(x_ref[...], w_ref[...],
                  preferred_element_type=jnp.float32)
    out = out + b_ref[...].astype(jnp.float32)
    pi_ref[...] = out[:, :cols].astype(pi_ref.dtype)
    vf_ref[...] = out[:, cols:].astype(vf_ref.dtype)


def _block_diag(w, g):
    f, c = w.shape
    eye = jnp.eye(g, dtype=w.dtype)
    return jnp.einsum('kj,fc->kfjc', eye, w).reshape(g * f, g * c)


def kernel(features, w_fused, b_fused):
    B, F = features.shape
    OUT = w_fused.shape[1]
    V = OUT - _P
    out_dtype = jnp.result_type(features.dtype, w_fused.dtype)
    b_fused = b_fused.reshape(OUT)

    g = _G if B % _G == 0 else 1
    w_big = jnp.concatenate(
        [_block_diag(w_fused[:, :_P], g), _block_diag(w_fused[:, _P:], g)],
        axis=1)
    b_big = jnp.concatenate(
        [jnp.tile(b_fused[:_P], g), jnp.tile(b_fused[_P:], g)]
    ).reshape(1, g * OUT)

    x_r = features.reshape(B // g, g * F)
    rows = B // g
    tbr = max(min(_TB // g, rows), 1)
    grid = (pl.cdiv(rows, tbr),)

    pi, vf = pl.pallas_call(
        _head_kernel,
        grid=grid,
        in_specs=[
            pl.BlockSpec((tbr, g * F), lambda i: (i, 0)),
            pl.BlockSpec((g * F, g * OUT), lambda i: (0, 0)),
            pl.BlockSpec((1, g * OUT), lambda i: (0, 0)),
        ],
        out_specs=[
            pl.BlockSpec((tbr, g * _P), lambda i: (i, 0)),
            pl.BlockSpec((tbr, g * V), lambda i: (i, 0)),
        ],
        out_shape=[
            jax.ShapeDtypeStruct((rows, g * _P), out_dtype),
            jax.ShapeDtypeStruct((rows, g * V), out_dtype),
        ],
        compiler_params=pltpu.CompilerParams(
            dimension_semantics=("parallel",),
            vmem_limit_bytes=64 << 20,
        ),
    )(x_r, w_big, b_big)
    return pi.reshape(B, _P), vf.reshape(B, V)


# speedup vs baseline: 6.0685x; 6.0685x over previous
import jax
import jax.numpy as jnp
from jax.experimental import pallas as pl
from jax.experimental.pallas import tpu as pltpu

_P = 4
_TB = 4096


def _head_kernel(x_ref, w_ref, b_ref, pi_ref, vf_ref):
    out = jnp.dot(x_ref[...], w_ref[...],
                  preferred_element_type=jnp.float32)
    out = out + b_ref[...].astype(jnp.float32)
    ot = out.T.astype(pi_ref.dtype)
    pi_ref[...] = ot[:_P, :]
    vf_ref[...] = ot[_P:, :]


def kernel(features, w_fused, b_fused):
    B, F = features.shape
    OUT = w_fused.shape[1]
    V = OUT - _P
    out_dtype = jnp.result_type(features.dtype, w_fused.dtype)
    b_fused = b_fused.reshape(1, OUT)

    tb = min(_TB, B)
    grid = (pl.cdiv(B, tb),)

    pi_t, vf_t = pl.pallas_call(
        _head_kernel,
        grid=grid,
        in_specs=[
            pl.BlockSpec((tb, F), lambda i: (i, 0)),
            pl.BlockSpec((F, OUT), lambda i: (0, 0)),
            pl.BlockSpec((1, OUT), lambda i: (0, 0)),
        ],
        out_specs=[
            pl.BlockSpec((_P, tb), lambda i: (0, i)),
            pl.BlockSpec((V, tb), lambda i: (0, i)),
        ],
        out_shape=[
            jax.ShapeDtypeStruct((_P, B), out_dtype),
            jax.ShapeDtypeStruct((V, B), out_dtype),
        ],
        compiler_params=pltpu.CompilerParams(
            dimension_semantics=("parallel",),
            vmem_limit_bytes=64 << 20,
        ),
    )(features, w_fused, b_fused)
    return pi_t.T, vf_t.T


# speedup vs baseline: 6.9989x; 1.1533x over previous
import jax
import jax.numpy as jnp
from jax.experimental import pallas as pl
from jax.experimental.pallas import tpu as pltpu

_P = 4
_TB = 8192


def _head_kernel(x_ref, w_ref, b_ref, pi_ref, vf_ref):
    out = jnp.dot(x_ref[...], w_ref[...],
                  preferred_element_type=jnp.float32)
    out = out + b_ref[...].astype(jnp.float32)
    ot = out.T.astype(pi_ref.dtype)
    pi_ref[...] = ot[:_P, :]
    vf_ref[...] = ot[_P:, :]


def kernel(features, w_fused, b_fused):
    B, F = features.shape
    OUT = w_fused.shape[1]
    V = OUT - _P
    out_dtype = jnp.result_type(features.dtype, w_fused.dtype)
    b_fused = b_fused.reshape(1, OUT)

    tb = min(_TB, B)
    grid = (pl.cdiv(B, tb),)

    pi_t, vf_t = pl.pallas_call(
        _head_kernel,
        grid=grid,
        in_specs=[
            pl.BlockSpec((tb, F), lambda i: (i, 0)),
            pl.BlockSpec((F, OUT), lambda i: (0, 0)),
            pl.BlockSpec((1, OUT), lambda i: (0, 0)),
        ],
        out_specs=[
            pl.BlockSpec((_P, tb), lambda i: (0, i)),
            pl.BlockSpec((V, tb), lambda i: (0, i)),
        ],
        out_shape=[
            jax.ShapeDtypeStruct((_P, B), out_dtype),
            jax.ShapeDtypeStruct((V, B), out_dtype),
        ],
        compiler_params=pltpu.CompilerParams(
            dimension_semantics=("parallel",),
            vmem_limit_bytes=64 << 20,
        ),
    )(features, w_fused, b_fused)
    return pi_t.T, vf_t.T
